# baseline (speedup 1.0000x reference)
import functools

import jax
import jax.numpy as jnp
from jax import lax
from jax.experimental import pallas as pl
from jax.experimental.pallas import tpu as pltpu
from jax.experimental.pallas import tpu_sc as plsc

N, C, H, W = 2, 96, 512, 512
HW = H * W
GW = 128

NC = 2
NS = 16
TROWS = 4096
NPASS = HW // TROWS
SRC_PER_SUB = HW // NS
SCHUNK = 1024
NCHUNK = SRC_PER_SUB // SCHUNK
TSUB = TROWS // NS
SLABS = ((0, 33), (33, 33), (66, 31))


def _prep_body(flow_ref, metric_ref, idx_ref, w_ref, *, rows_per_blk):
    r0 = pl.program_id(1) * rows_per_blk
    fx = flow_ref[0, 0]
    fy = flow_ref[0, 1]
    cols = lax.broadcasted_iota(jnp.int32, fx.shape, 1).astype(jnp.float32)
    rows = (lax.broadcasted_iota(jnp.int32, fx.shape, 0) + r0).astype(jnp.float32)
    xx = cols + fx
    yy = rows + fy
    x0f = jnp.floor(xx)
    y0f = jnp.floor(yy)
    x1f = x0f + 1.0
    y1f = y0f + 1.0
    expm = jnp.exp(metric_ref[0, 0])
    k = 0
    for x_f, y_f, wgt in (
        (x0f, y0f, (x1f - xx) * (y1f - yy)),
        (x1f, y0f, (xx - x0f) * (y1f - yy)),
        (x0f, y1f, (x1f - xx) * (yy - y0f)),
        (x1f, y1f, (xx - x0f) * (yy - y0f)),
    ):
        xl = x_f.astype(jnp.int32)
        yl = y_f.astype(jnp.int32)
        valid = (xl >= 0) & (xl < W) & (yl >= 0) & (yl < H)
        lin = yl * W + xl
        idx_ref[0, k] = jnp.where(valid, lin, jnp.int32(-1))
        w_ref[0, k] = wgt * valid.astype(jnp.float32) * expm
        k += 1


def _prep(tenFlow, tenMetric):
    rows_per_blk = 128
    grid = (N, H // rows_per_blk)
    idx, w = pl.pallas_call(
        functools.partial(_prep_body, rows_per_blk=rows_per_blk),
        grid=grid,
        in_specs=[
            pl.BlockSpec((1, 2, rows_per_blk, W), lambda n, r: (n, 0, r, 0)),
            pl.BlockSpec((1, 1, rows_per_blk, W), lambda n, r: (n, 0, r, 0)),
        ],
        out_specs=[
            pl.BlockSpec((1, 4, rows_per_blk, W), lambda n, r: (n, 0, r, 0)),
            pl.BlockSpec((1, 4, rows_per_blk, W), lambda n, r: (n, 0, r, 0)),
        ],
        out_shape=[
            jax.ShapeDtypeStruct((N, 4, H, W), jnp.int32),
            jax.ShapeDtypeStruct((N, 4, H, W), jnp.float32),
        ],
    )(tenFlow, tenMetric)
    return idx.reshape(N * 4, HW), w.reshape(N * 4, HW)


def _finalize_body(mid_ref, out_ref):
    x = mid_ref[0]
    y = jnp.transpose(x, (1, 0))
    norm = y[C]
    norm = jnp.where(norm == 0.0, jnp.float32(1.0), norm)
    out_ref[0] = y[:C] / norm[None, :]


def _finalize(mid):
    t = 2048
    grid = (N, HW // t)
    out = pl.pallas_call(
        _finalize_body,
        grid=grid,
        in_specs=[pl.BlockSpec((1, t, GW), lambda n, p: (n, p, 0))],
        out_specs=pl.BlockSpec((1, C, t), lambda n, p: (n, 0, p)),
        out_shape=jax.ShapeDtypeStruct((N, C, HW), jnp.float32),
    )(mid)
    return out.reshape(N, C, H, W)


def _sc_body(inp, idxh, wh, zeros_hbm, mid, table, idx_v, w_v, src_v, vals_v,
             idx128):
    cid = lax.axis_index("c")
    sid = lax.axis_index("s")

    def zero_vals_cols(c0, ncols):
        @pl.loop(0, ncols)
        def _zc(col):
            @pl.loop(0, 8)
            def _zi(i):
                plsc.store_scatter(
                    vals_v,
                    [i * 16 + lax.iota(jnp.int32, 16),
                     jnp.full((16,), jnp.int32(c0) + col, jnp.int32)],
                    jnp.zeros((16,), jnp.float32))

    zero_vals_cols(0, GW)

    def do_pass(n, base):
        pltpu.sync_copy(
            zeros_hbm,
            table.at[pl.ds(pl.multiple_of(sid * TSUB, TSUB), TSUB), :])
        plsc.subcore_barrier()

        @pl.loop(0, 4 * NCHUNK)
        def _q(qq):
            k = lax.div(qq, jnp.int32(NCHUNK))
            q = qq - k * NCHUNK
            a = n * 4 + k
            cr = pl.multiple_of(sid * (SRC_PER_SUB // 128) + q * 8, 8)
            pltpu.sync_copy(idxh.at[a, pl.ds(cr, 8), :], idx_v)
            pltpu.sync_copy(wh.at[a, pl.ds(cr, 8), :], w_v)

            for si, (c0, nch) in enumerate(SLABS):
                with_metric = si == len(SLABS) - 1
                for jc in range(nch):
                    pltpu.sync_copy(inp.at[n, c0 + jc, pl.ds(cr, 8), :],
                                    src_v.at[pl.ds(jc * 8, 8), :])
                prev0, prevn = SLABS[si - 1]
                pn = prevn + (1 if si == 0 else 0)

                @pl.loop(0, 8)
                def _batch(j, c0=c0, nch=nch, with_metric=with_metric,
                           prev0=prev0, pn=pn):
                    zero_vals_cols(prev0, pn)

                    @pl.loop(0, 8)
                    def _compute(i):
                        rows = i * 16 + lax.iota(jnp.int32, 16)
                        w16 = w_v[j, pl.ds(i * 16, 16)]
                        raw = idx_v[j, pl.ds(i * 16, 16)]
                        loc = raw - base
                        ok = (raw >= base) & (raw < base + TROWS)
                        idx128[pl.ds(i * 16, 16)] = jnp.where(
                            ok, loc, jnp.int32(-1))
                        for jc in range(nch):
                            x = src_v[jc * 8 + j, pl.ds(i * 16, 16)] * w16
                            plsc.store_scatter(
                                vals_v,
                                [rows, jnp.full((16,), c0 + jc, jnp.int32)], x)
                        if with_metric:
                            plsc.store_scatter(
                                vals_v, [rows, jnp.full((16,), C, jnp.int32)],
                                w16)

                    pltpu.sync_copy(
                        vals_v,
                        table.at[plsc.Indices(idx128, ignored_value=-1)],
                        add=True)

        plsc.subcore_barrier()
        pltpu.sync_copy(
            table.at[pl.ds(pl.multiple_of(sid * TSUB, TSUB), TSUB), :],
            mid.at[n, pl.ds(pl.multiple_of(base + sid * TSUB, TSUB), TSUB), :])
        plsc.subcore_barrier()

    @pl.loop(cid, 2 * NPASS, step=NC)
    def _p(t):
        n = lax.div(t, jnp.int32(NPASS))
        base = (t - n * NPASS) * TROWS
        do_pass(n, base)


def _scatter_sc(inp, idx, w):
    mesh = plsc.VectorSubcoreMesh(
        core_axis_name="c", subcore_axis_name="s", num_cores=NC, num_subcores=NS)
    f = pl.kernel(
        _sc_body,
        out_type=jax.ShapeDtypeStruct((N, HW, GW), jnp.float32),
        mesh=mesh,
        compiler_params=pltpu.CompilerParams(needs_layout_passes=False),
        scratch_types=[
            pltpu.VMEM_SHARED((TROWS, GW), jnp.float32),
            pltpu.VMEM((8, 128), jnp.int32),
            pltpu.VMEM((8, 128), jnp.float32),
            pltpu.VMEM((33 * 8, 128), jnp.float32),
            pltpu.VMEM((128, GW), jnp.float32),
            pltpu.VMEM((128,), jnp.int32),
        ],
    )
    zeros_hbm = jnp.zeros((TSUB, GW), jnp.float32)
    return f(inp.reshape(N, C, HW // 128, 128),
             idx.reshape(N * 4, HW // 128, 128),
             w.reshape(N * 4, HW // 128, 128),
             zeros_hbm)


def kernel(tenInput, tenFlow, tenMetric):
    idx, w = _prep(tenFlow, tenMetric)
    inp = tenInput.reshape(N, C, HW)
    mid = _scatter_sc(inp, idx, w)
    return _finalize(mid)


# speedup vs baseline: 2.4113x; 2.4113x over previous
import functools

import jax
import jax.numpy as jnp
from jax import lax
from jax.experimental import pallas as pl
from jax.experimental.pallas import tpu as pltpu
from jax.experimental.pallas import tpu_sc as plsc

N, C, H, W = 2, 96, 512, 512
HW = H * W
GW = 128

NC = 2
NS = 16
TROWS = 8192
NPASS = HW // TROWS
SRC_PER_SUB = HW // NS
SCHUNK = 1024
NCHUNK = SRC_PER_SUB // SCHUNK
TSUB = TROWS // NS
SLABS = ((0, 17), (17, 17), (34, 17), (51, 17), (68, 17), (85, 11))


def _prep_body(flow_ref, metric_ref, idx_ref, w_ref, *, rows_per_blk):
    r0 = pl.program_id(1) * rows_per_blk
    fx = flow_ref[0, 0]
    fy = flow_ref[0, 1]
    cols = lax.broadcasted_iota(jnp.int32, fx.shape, 1).astype(jnp.float32)
    rows = (lax.broadcasted_iota(jnp.int32, fx.shape, 0) + r0).astype(jnp.float32)
    xx = cols + fx
    yy = rows + fy
    x0f = jnp.floor(xx)
    y0f = jnp.floor(yy)
    x1f = x0f + 1.0
    y1f = y0f + 1.0
    expm = jnp.exp(metric_ref[0, 0])
    k = 0
    for x_f, y_f, wgt in (
        (x0f, y0f, (x1f - xx) * (y1f - yy)),
        (x1f, y0f, (xx - x0f) * (y1f - yy)),
        (x0f, y1f, (x1f - xx) * (yy - y0f)),
        (x1f, y1f, (xx - x0f) * (yy - y0f)),
    ):
        xl = x_f.astype(jnp.int32)
        yl = y_f.astype(jnp.int32)
        valid = (xl >= 0) & (xl < W) & (yl >= 0) & (yl < H)
        lin = yl * W + xl
        idx_ref[0, k] = jnp.where(valid, lin, jnp.int32(-1))
        w_ref[0, k] = wgt * valid.astype(jnp.float32) * expm
        k += 1


def _prep(tenFlow, tenMetric):
    rows_per_blk = 128
    grid = (N, H // rows_per_blk)
    idx, w = pl.pallas_call(
        functools.partial(_prep_body, rows_per_blk=rows_per_blk),
        grid=grid,
        in_specs=[
            pl.BlockSpec((1, 2, rows_per_blk, W), lambda n, r: (n, 0, r, 0)),
            pl.BlockSpec((1, 1, rows_per_blk, W), lambda n, r: (n, 0, r, 0)),
        ],
        out_specs=[
            pl.BlockSpec((1, 4, rows_per_blk, W), lambda n, r: (n, 0, r, 0)),
            pl.BlockSpec((1, 4, rows_per_blk, W), lambda n, r: (n, 0, r, 0)),
        ],
        out_shape=[
            jax.ShapeDtypeStruct((N, 4, H, W), jnp.int32),
            jax.ShapeDtypeStruct((N, 4, H, W), jnp.float32),
        ],
    )(tenFlow, tenMetric)
    return idx.reshape(N * 4, HW), w.reshape(N * 4, HW)


def _finalize_body(mid_ref, out_ref):
    x = mid_ref[0]
    y = jnp.transpose(x, (1, 0))
    norm = y[C]
    norm = jnp.where(norm == 0.0, jnp.float32(1.0), norm)
    out_ref[0] = y[:C] / norm[None, :]


def _finalize(mid):
    t = 2048
    grid = (N, HW // t)
    out = pl.pallas_call(
        _finalize_body,
        grid=grid,
        in_specs=[pl.BlockSpec((1, t, GW), lambda n, p: (n, p, 0))],
        out_specs=pl.BlockSpec((1, C, t), lambda n, p: (n, 0, p)),
        out_shape=jax.ShapeDtypeStruct((N, C, HW), jnp.float32),
    )(mid)
    return out.reshape(N, C, H, W)


def _sc_body(inp, idxh, wh, zeros_hbm, mid, table, idx_v, w_v, src_v, vals_v,
             idx128):
    cid = lax.axis_index("c")
    sid = lax.axis_index("s")

    def zero_vals_cols(c0, ncols):
        @pl.loop(0, ncols)
        def _zc(col):
            @pl.loop(0, 8)
            def _zi(i):
                plsc.store_scatter(
                    vals_v,
                    [i * 16 + lax.iota(jnp.int32, 16),
                     jnp.full((16,), jnp.int32(c0) + col, jnp.int32)],
                    jnp.zeros((16,), jnp.float32))

    zero_vals_cols(0, GW)

    def do_pass(n, base):
        pltpu.sync_copy(
            zeros_hbm,
            table.at[pl.ds(pl.multiple_of(sid * TSUB, TSUB), TSUB), :])
        plsc.subcore_barrier()

        @pl.loop(0, 4 * NCHUNK)
        def _q(qq):
            k = lax.div(qq, jnp.int32(NCHUNK))
            q = qq - k * NCHUNK
            a = n * 4 + k
            cr = pl.multiple_of(sid * (SRC_PER_SUB // 128) + q * 8, 8)
            pltpu.sync_copy(idxh.at[a, pl.ds(cr, 8), :], idx_v)
            pltpu.sync_copy(wh.at[a, pl.ds(cr, 8), :], w_v)

            for si, (c0, nch) in enumerate(SLABS):
                with_metric = si == len(SLABS) - 1
                for jc in range(nch):
                    pltpu.sync_copy(inp.at[n, c0 + jc, pl.ds(cr, 8), :],
                                    src_v.at[pl.ds(jc * 8, 8), :])
                prev0, prevn = SLABS[si - 1]
                pn = prevn + (1 if si == 0 else 0)
                zero_vals_cols(prev0, pn)

                @pl.loop(0, 8)
                def _batch(j, c0=c0, nch=nch, with_metric=with_metric):
                    @pl.loop(0, 8)
                    def _compute(i):
                        rows = i * 16 + lax.iota(jnp.int32, 16)
                        w16 = w_v[j, pl.ds(i * 16, 16)]
                        raw = idx_v[j, pl.ds(i * 16, 16)]
                        loc = raw - base
                        ok = (raw >= base) & (raw < base + TROWS)
                        idx128[pl.ds(i * 16, 16)] = jnp.where(
                            ok, loc, jnp.int32(-1))
                        for jc in range(nch):
                            x = src_v[jc * 8 + j, pl.ds(i * 16, 16)] * w16
                            plsc.store_scatter(
                                vals_v,
                                [rows, jnp.full((16,), c0 + jc, jnp.int32)], x)
                        if with_metric:
                            plsc.store_scatter(
                                vals_v, [rows, jnp.full((16,), C, jnp.int32)],
                                w16)

                    pltpu.sync_copy(
                        vals_v,
                        table.at[plsc.Indices(idx128, ignored_value=-1)],
                        add=True)

        plsc.subcore_barrier()
        pltpu.sync_copy(
            table.at[pl.ds(pl.multiple_of(sid * TSUB, TSUB), TSUB), :],
            mid.at[n, pl.ds(pl.multiple_of(base + sid * TSUB, TSUB), TSUB), :])
        plsc.subcore_barrier()

    @pl.loop(cid, 2 * NPASS, step=NC)
    def _p(t):
        n = lax.div(t, jnp.int32(NPASS))
        base = (t - n * NPASS) * TROWS
        do_pass(n, base)


def _scatter_sc(inp, idx, w):
    mesh = plsc.VectorSubcoreMesh(
        core_axis_name="c", subcore_axis_name="s", num_cores=NC, num_subcores=NS)
    f = pl.kernel(
        _sc_body,
        out_type=jax.ShapeDtypeStruct((N, HW, GW), jnp.float32),
        mesh=mesh,
        compiler_params=pltpu.CompilerParams(needs_layout_passes=False),
        scratch_types=[
            pltpu.VMEM_SHARED((TROWS, GW), jnp.float32),
            pltpu.VMEM((8, 128), jnp.int32),
            pltpu.VMEM((8, 128), jnp.float32),
            pltpu.VMEM((17 * 8, 128), jnp.float32),
            pltpu.VMEM((128, GW), jnp.float32),
            pltpu.VMEM((128,), jnp.int32),
        ],
    )
    zeros_hbm = jnp.zeros((TSUB, GW), jnp.float32)
    return f(inp.reshape(N, C, HW // 128, 128),
             idx.reshape(N * 4, HW // 128, 128),
             w.reshape(N * 4, HW // 128, 128),
             zeros_hbm)


def kernel(tenInput, tenFlow, tenMetric):
    idx, w = _prep(tenFlow, tenMetric)
    inp = tenInput.reshape(N, C, HW)
    mid = _scatter_sc(inp, idx, w)
    return _finalize(mid)


# speedup vs baseline: 3.4069x; 1.4129x over previous
import functools

import jax
import jax.numpy as jnp
from jax import lax
from jax.experimental import pallas as pl
from jax.experimental.pallas import tpu as pltpu
from jax.experimental.pallas import tpu_sc as plsc

N, C, H, W = 2, 96, 512, 512
HW = H * W
GW = 128

NC = 2
NS = 16
TROWS = 8192
NPASS = HW // TROWS
SRC_PER_SUB = HW // NS
SCHUNK = 1024
NCHUNK = SRC_PER_SUB // SCHUNK
TSUB = TROWS // NS
SLABS = ((0, 17), (17, 17), (34, 17), (51, 17), (68, 17), (85, 11))


def _prep_body(flow_ref, metric_ref, idx_ref, w_ref, *, rows_per_blk):
    r0 = pl.program_id(1) * rows_per_blk
    fx = flow_ref[0, 0]
    fy = flow_ref[0, 1]
    cols = lax.broadcasted_iota(jnp.int32, fx.shape, 1).astype(jnp.float32)
    rows = (lax.broadcasted_iota(jnp.int32, fx.shape, 0) + r0).astype(jnp.float32)
    xx = cols + fx
    yy = rows + fy
    x0f = jnp.floor(xx)
    y0f = jnp.floor(yy)
    x1f = x0f + 1.0
    y1f = y0f + 1.0
    expm = jnp.exp(metric_ref[0, 0])
    k = 0
    for x_f, y_f, wgt in (
        (x0f, y0f, (x1f - xx) * (y1f - yy)),
        (x1f, y0f, (xx - x0f) * (y1f - yy)),
        (x0f, y1f, (x1f - xx) * (yy - y0f)),
        (x1f, y1f, (xx - x0f) * (yy - y0f)),
    ):
        xl = x_f.astype(jnp.int32)
        yl = y_f.astype(jnp.int32)
        valid = (xl >= 0) & (xl < W) & (yl >= 0) & (yl < H)
        lin = yl * W + xl
        idx_ref[0, k] = jnp.where(valid, lin, jnp.int32(-1))
        w_ref[0, k] = wgt * valid.astype(jnp.float32) * expm
        k += 1


def _prep(tenFlow, tenMetric):
    rows_per_blk = 128
    grid = (N, H // rows_per_blk)
    idx, w = pl.pallas_call(
        functools.partial(_prep_body, rows_per_blk=rows_per_blk),
        grid=grid,
        in_specs=[
            pl.BlockSpec((1, 2, rows_per_blk, W), lambda n, r: (n, 0, r, 0)),
            pl.BlockSpec((1, 1, rows_per_blk, W), lambda n, r: (n, 0, r, 0)),
        ],
        out_specs=[
            pl.BlockSpec((1, 4, rows_per_blk, W), lambda n, r: (n, 0, r, 0)),
            pl.BlockSpec((1, 4, rows_per_blk, W), lambda n, r: (n, 0, r, 0)),
        ],
        out_shape=[
            jax.ShapeDtypeStruct((N, 4, H, W), jnp.int32),
            jax.ShapeDtypeStruct((N, 4, H, W), jnp.float32),
        ],
    )(tenFlow, tenMetric)
    return idx.reshape(N * 4, HW), w.reshape(N * 4, HW)


def _finalize_body(mid_ref, out_ref):
    x = mid_ref[0]
    y = jnp.transpose(x, (1, 0))
    norm = y[C]
    norm = jnp.where(norm == 0.0, jnp.float32(1.0), norm)
    out_ref[0] = y[:C] / norm[None, :]


def _finalize(mid):
    t = 2048
    grid = (N, HW // t)
    out = pl.pallas_call(
        _finalize_body,
        grid=grid,
        in_specs=[pl.BlockSpec((1, t, GW), lambda n, p: (n, p, 0))],
        out_specs=pl.BlockSpec((1, C, t), lambda n, p: (n, 0, p)),
        out_shape=jax.ShapeDtypeStruct((N, C, HW), jnp.float32),
    )(mid)
    return out.reshape(N, C, H, W)


def _sc_body(inp, idxh, wh, zeros_hbm, mid, table, idx_v, w_v, src_v, vals_v,
             idx128, dma_sem):
    cid = lax.axis_index("c")
    sid = lax.axis_index("s")

    def zero_vals_cols(c0, ncols):
        @pl.loop(0, ncols)
        def _zc(col):
            @pl.loop(0, 8)
            def _zi(i):
                plsc.store_scatter(
                    vals_v,
                    [i * 16 + lax.iota(jnp.int32, 16),
                     jnp.full((16,), jnp.int32(c0) + col, jnp.int32)],
                    jnp.zeros((16,), jnp.float32))

    zero_vals_cols(0, GW)

    def do_pass(n, base):
        pltpu.sync_copy(
            zeros_hbm,
            table.at[pl.ds(pl.multiple_of(sid * TSUB, TSUB), TSUB), :])
        plsc.subcore_barrier()

        @pl.loop(0, 4 * NCHUNK)
        def _q(qq):
            k = lax.div(qq, jnp.int32(NCHUNK))
            q = qq - k * NCHUNK
            a = n * 4 + k
            cr = pl.multiple_of(sid * (SRC_PER_SUB // 128) + q * 8, 8)
            d_idx = pltpu.async_copy(idxh.at[a, pl.ds(cr, 8), :], idx_v,
                                     dma_sem)
            d_w = pltpu.async_copy(wh.at[a, pl.ds(cr, 8), :], w_v, dma_sem)
            d_idx.wait()
            d_w.wait()

            for si, (c0, nch) in enumerate(SLABS):
                with_metric = si == len(SLABS) - 1
                descs = [
                    pltpu.async_copy(inp.at[n, c0 + jc, pl.ds(cr, 8), :],
                                     src_v.at[pl.ds(jc * 8, 8), :], dma_sem)
                    for jc in range(nch)
                ]
                for d in descs:
                    d.wait()
                prev0, prevn = SLABS[si - 1]
                pn = prevn + (1 if si == 0 else 0)
                zero_vals_cols(prev0, pn)

                @pl.loop(0, 8)
                def _batch(j, c0=c0, nch=nch, with_metric=with_metric):
                    @pl.loop(0, 8)
                    def _compute(i):
                        rows = i * 16 + lax.iota(jnp.int32, 16)
                        w16 = w_v[j, pl.ds(i * 16, 16)]
                        raw = idx_v[j, pl.ds(i * 16, 16)]
                        loc = raw - base
                        ok = (raw >= base) & (raw < base + TROWS)
                        idx128[pl.ds(i * 16, 16)] = jnp.where(
                            ok, loc, jnp.int32(-1))
                        for jc in range(nch):
                            x = src_v[jc * 8 + j, pl.ds(i * 16, 16)] * w16
                            plsc.store_scatter(
                                vals_v,
                                [rows, jnp.full((16,), c0 + jc, jnp.int32)], x)
                        if with_metric:
                            plsc.store_scatter(
                                vals_v, [rows, jnp.full((16,), C, jnp.int32)],
                                w16)

                    pltpu.sync_copy(
                        vals_v,
                        table.at[plsc.Indices(idx128, ignored_value=-1)],
                        add=True)

        plsc.subcore_barrier()
        pltpu.sync_copy(
            table.at[pl.ds(pl.multiple_of(sid * TSUB, TSUB), TSUB), :],
            mid.at[n, pl.ds(pl.multiple_of(base + sid * TSUB, TSUB), TSUB), :])
        plsc.subcore_barrier()

    @pl.loop(cid, 2 * NPASS, step=NC)
    def _p(t):
        n = lax.div(t, jnp.int32(NPASS))
        base = (t - n * NPASS) * TROWS
        do_pass(n, base)


def _scatter_sc(inp, idx, w):
    mesh = plsc.VectorSubcoreMesh(
        core_axis_name="c", subcore_axis_name="s", num_cores=NC, num_subcores=NS)
    f = pl.kernel(
        _sc_body,
        out_type=jax.ShapeDtypeStruct((N, HW, GW), jnp.float32),
        mesh=mesh,
        compiler_params=pltpu.CompilerParams(needs_layout_passes=False),
        scratch_types=[
            pltpu.VMEM_SHARED((TROWS, GW), jnp.float32),
            pltpu.VMEM((8, 128), jnp.int32),
            pltpu.VMEM((8, 128), jnp.float32),
            pltpu.VMEM((17 * 8, 128), jnp.float32),
            pltpu.VMEM((128, GW), jnp.float32),
            pltpu.VMEM((128,), jnp.int32),
            pltpu.SemaphoreType.DMA,
        ],
    )
    zeros_hbm = jnp.zeros((TSUB, GW), jnp.float32)
    return f(inp.reshape(N, C, HW // 128, 128),
             idx.reshape(N * 4, HW // 128, 128),
             w.reshape(N * 4, HW // 128, 128),
             zeros_hbm)


def kernel(tenInput, tenFlow, tenMetric):
    idx, w = _prep(tenFlow, tenMetric)
    inp = tenInput.reshape(N, C, HW)
    mid = _scatter_sc(inp, idx, w)
    return _finalize(mid)


# speedup vs baseline: 3.7737x; 1.1077x over previous
import functools

import jax
import jax.numpy as jnp
from jax import lax
from jax.experimental import pallas as pl
from jax.experimental.pallas import tpu as pltpu
from jax.experimental.pallas import tpu_sc as plsc

N, C, H, W = 2, 96, 512, 512
HW = H * W
GW = 128

NC = 2
NS = 16
TROWS = 8192
NPASS = HW // TROWS
SRC_PER_SUB = HW // NS
SCHUNK = 1024
NCHUNK = SRC_PER_SUB // SCHUNK
TSUB = TROWS // NS
SLABS = ((0, 17), (17, 17), (34, 17), (51, 17), (68, 17), (85, 11))


def _prep_body(flow_ref, metric_ref, idx_ref, w_ref, *, rows_per_blk):
    r0 = pl.program_id(1) * rows_per_blk
    fx = flow_ref[0, 0]
    fy = flow_ref[0, 1]
    cols = lax.broadcasted_iota(jnp.int32, fx.shape, 1).astype(jnp.float32)
    rows = (lax.broadcasted_iota(jnp.int32, fx.shape, 0) + r0).astype(jnp.float32)
    xx = cols + fx
    yy = rows + fy
    x0f = jnp.floor(xx)
    y0f = jnp.floor(yy)
    x1f = x0f + 1.0
    y1f = y0f + 1.0
    expm = jnp.exp(metric_ref[0, 0])
    k = 0
    for x_f, y_f, wgt in (
        (x0f, y0f, (x1f - xx) * (y1f - yy)),
        (x1f, y0f, (xx - x0f) * (y1f - yy)),
        (x0f, y1f, (x1f - xx) * (yy - y0f)),
        (x1f, y1f, (xx - x0f) * (yy - y0f)),
    ):
        xl = x_f.astype(jnp.int32)
        yl = y_f.astype(jnp.int32)
        valid = (xl >= 0) & (xl < W) & (yl >= 0) & (yl < H)
        lin = yl * W + xl
        idx_ref[0, k] = jnp.where(valid, lin, jnp.int32(-1))
        w_ref[0, k] = wgt * valid.astype(jnp.float32) * expm
        k += 1


def _prep(tenFlow, tenMetric):
    rows_per_blk = 128
    grid = (N, H // rows_per_blk)
    idx, w = pl.pallas_call(
        functools.partial(_prep_body, rows_per_blk=rows_per_blk),
        grid=grid,
        in_specs=[
            pl.BlockSpec((1, 2, rows_per_blk, W), lambda n, r: (n, 0, r, 0)),
            pl.BlockSpec((1, 1, rows_per_blk, W), lambda n, r: (n, 0, r, 0)),
        ],
        out_specs=[
            pl.BlockSpec((1, 4, rows_per_blk, W), lambda n, r: (n, 0, r, 0)),
            pl.BlockSpec((1, 4, rows_per_blk, W), lambda n, r: (n, 0, r, 0)),
        ],
        out_shape=[
            jax.ShapeDtypeStruct((N, 4, H, W), jnp.int32),
            jax.ShapeDtypeStruct((N, 4, H, W), jnp.float32),
        ],
    )(tenFlow, tenMetric)
    return idx.reshape(N * 4, HW), w.reshape(N * 4, HW)


def _finalize_body(mid_ref, out_ref):
    x = mid_ref[0]
    y = jnp.transpose(x, (1, 0))
    norm = y[C]
    norm = jnp.where(norm == 0.0, jnp.float32(1.0), norm)
    out_ref[0] = y[:C] / norm[None, :]


def _finalize(mid):
    t = 2048
    grid = (N, HW // t)
    out = pl.pallas_call(
        _finalize_body,
        grid=grid,
        in_specs=[pl.BlockSpec((1, t, GW), lambda n, p: (n, p, 0))],
        out_specs=pl.BlockSpec((1, C, t), lambda n, p: (n, 0, p)),
        out_shape=jax.ShapeDtypeStruct((N, C, HW), jnp.float32),
    )(mid)
    return out.reshape(N, C, H, W)


def _sc_body(inp, idxh, wh, zeros_hbm, mid, table, idx_v, w_v, src_v, vals_v,
             idx128, dma_sem):
    cid = lax.axis_index("c")
    sid = lax.axis_index("s")

    def zero_vals_cols(c0, ncols):
        @pl.loop(0, ncols)
        def _zc(col):
            @pl.loop(0, 8)
            def _zi(i):
                plsc.store_scatter(
                    vals_v,
                    [i * 16 + lax.iota(jnp.int32, 16),
                     jnp.full((16,), jnp.int32(c0) + col, jnp.int32)],
                    jnp.zeros((16,), jnp.float32))

    zero_vals_cols(0, GW)

    def do_pass(n, base):
        pltpu.sync_copy(
            zeros_hbm,
            table.at[pl.ds(pl.multiple_of(sid * TSUB, TSUB), TSUB), :])
        plsc.subcore_barrier()

        @pl.loop(0, 4 * NCHUNK)
        def _q(qq):
            k = lax.div(qq, jnp.int32(NCHUNK))
            q = qq - k * NCHUNK
            a = n * 4 + k
            cr = pl.multiple_of(sid * (SRC_PER_SUB // 128) + q * 8, 8)
            d_idx = pltpu.async_copy(idxh.at[a, pl.ds(cr, 8), :], idx_v,
                                     dma_sem)
            d_w = pltpu.async_copy(wh.at[a, pl.ds(cr, 8), :], w_v, dma_sem)
            d_idx.wait()
            d_w.wait()

            for si, (c0, nch) in enumerate(SLABS):
                with_metric = si == len(SLABS) - 1
                descs = [
                    pltpu.async_copy(inp.at[n, c0 + jc, pl.ds(cr, 8), :],
                                     src_v.at[pl.ds(jc * 8, 8), :], dma_sem)
                    for jc in range(nch)
                ]
                for d in descs:
                    d.wait()
                prev0, prevn = SLABS[si - 1]
                pn = prevn + (1 if si == 0 else 0)
                zero_vals_cols(prev0, pn)

                @pl.loop(0, 8)
                def _batch(j, c0=c0, nch=nch, with_metric=with_metric):
                    @pl.loop(0, 8)
                    def _compute(i):
                        rows = i * 16 + lax.iota(jnp.int32, 16)
                        w16 = w_v[j, pl.ds(i * 16, 16)]
                        raw = idx_v[j, pl.ds(i * 16, 16)]
                        loc = raw - base
                        ok = (raw >= base) & (raw < base + TROWS)
                        idx128[pl.ds(i * 16, 16)] = jnp.where(
                            ok, loc, jnp.int32(-1))
                        for jc in range(nch):
                            x = src_v[jc * 8 + j, pl.ds(i * 16, 16)] * w16
                            plsc.store_scatter(
                                vals_v,
                                [rows, jnp.full((16,), c0 + jc, jnp.int32)], x)
                        if with_metric:
                            plsc.store_scatter(
                                vals_v, [rows, jnp.full((16,), C, jnp.int32)],
                                w16)

                    def _mx(i, m):
                        v = idx128[pl.ds(i * 16, 16)]
                        return jnp.maximum(m, lax.reduce_max(v, (0,)))

                    hit = lax.fori_loop(0, 8, _mx, jnp.int32(-1))

                    @pl.when(hit >= 0)
                    def _do_scatter():
                        pltpu.sync_copy(
                            vals_v,
                            table.at[plsc.Indices(idx128, ignored_value=-1)],
                            add=True)

        plsc.subcore_barrier()
        pltpu.sync_copy(
            table.at[pl.ds(pl.multiple_of(sid * TSUB, TSUB), TSUB), :],
            mid.at[n, pl.ds(pl.multiple_of(base + sid * TSUB, TSUB), TSUB), :])
        plsc.subcore_barrier()

    @pl.loop(cid, 2 * NPASS, step=NC)
    def _p(t):
        n = lax.div(t, jnp.int32(NPASS))
        base = (t - n * NPASS) * TROWS
        do_pass(n, base)


def _scatter_sc(inp, idx, w):
    mesh = plsc.VectorSubcoreMesh(
        core_axis_name="c", subcore_axis_name="s", num_cores=NC, num_subcores=NS)
    f = pl.kernel(
        _sc_body,
        out_type=jax.ShapeDtypeStruct((N, HW, GW), jnp.float32),
        mesh=mesh,
        compiler_params=pltpu.CompilerParams(needs_layout_passes=False),
        scratch_types=[
            pltpu.VMEM_SHARED((TROWS, GW), jnp.float32),
            pltpu.VMEM((8, 128), jnp.int32),
            pltpu.VMEM((8, 128), jnp.float32),
            pltpu.VMEM((17 * 8, 128), jnp.float32),
            pltpu.VMEM((128, GW), jnp.float32),
            pltpu.VMEM((128,), jnp.int32),
            pltpu.SemaphoreType.DMA,
        ],
    )
    zeros_hbm = jnp.zeros((TSUB, GW), jnp.float32)
    return f(inp.reshape(N, C, HW // 128, 128),
             idx.reshape(N * 4, HW // 128, 128),
             w.reshape(N * 4, HW // 128, 128),
             zeros_hbm)


def kernel(tenInput, tenFlow, tenMetric):
    idx, w = _prep(tenFlow, tenMetric)
    inp = tenInput.reshape(N, C, HW)
    mid = _scatter_sc(inp, idx, w)
    return _finalize(mid)


# speedup vs baseline: 5.1978x; 1.3774x over previous
import functools

import jax
import jax.numpy as jnp
from jax import lax
from jax.experimental import pallas as pl
from jax.experimental.pallas import tpu as pltpu
from jax.experimental.pallas import tpu_sc as plsc

N, C, H, W = 2, 96, 512, 512
HW = H * W
GW = 128

NC = 2
NS = 16
TROWS = 8192
NPASS = HW // TROWS
SRC_PER_SUB = HW // NS
SCHUNK = 1024
NCHUNK = SRC_PER_SUB // SCHUNK
TSUB = TROWS // NS
SLABS = ((0, 17), (17, 17), (34, 17), (51, 17), (68, 17), (85, 11))


def _prep_body(flow_ref, metric_ref, idx_ref, w_ref, *, rows_per_blk):
    r0 = pl.program_id(1) * rows_per_blk
    fx = flow_ref[0, 0]
    fy = flow_ref[0, 1]
    cols = lax.broadcasted_iota(jnp.int32, fx.shape, 1).astype(jnp.float32)
    rows = (lax.broadcasted_iota(jnp.int32, fx.shape, 0) + r0).astype(jnp.float32)
    xx = cols + fx
    yy = rows + fy
    x0f = jnp.floor(xx)
    y0f = jnp.floor(yy)
    x1f = x0f + 1.0
    y1f = y0f + 1.0
    expm = jnp.exp(metric_ref[0, 0])
    k = 0
    for x_f, y_f, wgt in (
        (x0f, y0f, (x1f - xx) * (y1f - yy)),
        (x1f, y0f, (xx - x0f) * (y1f - yy)),
        (x0f, y1f, (x1f - xx) * (yy - y0f)),
        (x1f, y1f, (xx - x0f) * (yy - y0f)),
    ):
        xl = x_f.astype(jnp.int32)
        yl = y_f.astype(jnp.int32)
        valid = (xl >= 0) & (xl < W) & (yl >= 0) & (yl < H)
        lin = yl * W + xl
        idx_ref[0, k] = jnp.where(valid, lin, jnp.int32(-1))
        w_ref[0, k] = wgt * valid.astype(jnp.float32) * expm
        k += 1


def _prep(tenFlow, tenMetric):
    rows_per_blk = 128
    grid = (N, H // rows_per_blk)
    idx, w = pl.pallas_call(
        functools.partial(_prep_body, rows_per_blk=rows_per_blk),
        grid=grid,
        in_specs=[
            pl.BlockSpec((1, 2, rows_per_blk, W), lambda n, r: (n, 0, r, 0)),
            pl.BlockSpec((1, 1, rows_per_blk, W), lambda n, r: (n, 0, r, 0)),
        ],
        out_specs=[
            pl.BlockSpec((1, 4, rows_per_blk, W), lambda n, r: (n, 0, r, 0)),
            pl.BlockSpec((1, 4, rows_per_blk, W), lambda n, r: (n, 0, r, 0)),
        ],
        out_shape=[
            jax.ShapeDtypeStruct((N, 4, H, W), jnp.int32),
            jax.ShapeDtypeStruct((N, 4, H, W), jnp.float32),
        ],
    )(tenFlow, tenMetric)
    return idx.reshape(N * 4, HW), w.reshape(N * 4, HW)


def _finalize_body(mid_ref, out_ref):
    x = mid_ref[0]
    y = jnp.transpose(x, (1, 0))
    norm = y[C]
    norm = jnp.where(norm == 0.0, jnp.float32(1.0), norm)
    out_ref[0] = y[:C] / norm[None, :]


def _finalize(mid):
    t = 2048
    grid = (N, HW // t)
    out = pl.pallas_call(
        _finalize_body,
        grid=grid,
        in_specs=[pl.BlockSpec((1, t, GW), lambda n, p: (n, p, 0))],
        out_specs=pl.BlockSpec((1, C, t), lambda n, p: (n, 0, p)),
        out_shape=jax.ShapeDtypeStruct((N, C, HW), jnp.float32),
    )(mid)
    return out.reshape(N, C, H, W)


def _sc_body(inp, idxh, wh, zeros_hbm, mid, table, idx_v, w_v, src_v, vals_v,
             idx128, dma_sem):
    cid = lax.axis_index("c")
    sid = lax.axis_index("s")

    def zero_vals_cols(c0, ncols):
        @pl.loop(0, ncols)
        def _zc(col):
            @pl.loop(0, 8)
            def _zi(i):
                plsc.store_scatter(
                    vals_v,
                    [i * 16 + lax.iota(jnp.int32, 16),
                     jnp.full((16,), jnp.int32(c0) + col, jnp.int32)],
                    jnp.zeros((16,), jnp.float32))

    zero_vals_cols(0, GW)

    def do_pass(n, base):
        pltpu.sync_copy(
            zeros_hbm,
            table.at[pl.ds(pl.multiple_of(sid * TSUB, TSUB), TSUB), :])
        plsc.subcore_barrier()

        @pl.loop(0, 4 * NCHUNK)
        def _q(qq):
            k = lax.div(qq, jnp.int32(NCHUNK))
            q = qq - k * NCHUNK
            a = n * 4 + k
            cr = pl.multiple_of(sid * (SRC_PER_SUB // 128) + q * 8, 8)
            d_idx = pltpu.async_copy(idxh.at[a, pl.ds(cr, 8), :], idx_v,
                                     dma_sem)
            d_w = pltpu.async_copy(wh.at[a, pl.ds(cr, 8), :], w_v, dma_sem)
            d_idx.wait()
            d_w.wait()

            for si, (c0, nch) in enumerate(SLABS):
                with_metric = si == len(SLABS) - 1
                descs = [
                    pltpu.async_copy(inp.at[n, c0 + jc, pl.ds(cr, 8), :],
                                     src_v.at[pl.ds(jc * 8, 8), :], dma_sem)
                    for jc in range(nch)
                ]
                for d in descs:
                    d.wait()
                prev0, prevn = SLABS[si - 1]
                pn = prevn + (1 if si == 0 else 0)
                zero_vals_cols(prev0, pn)

                @pl.loop(0, 8)
                def _batch(j, c0=c0, nch=nch, with_metric=with_metric):
                    def _flt(i, m):
                        raw = idx_v[j, pl.ds(i * 16, 16)]
                        loc = raw - base
                        ok = (raw >= base) & (raw < base + TROWS)
                        v = jnp.where(ok, loc, jnp.int32(-1))
                        idx128[pl.ds(i * 16, 16)] = v
                        return jnp.maximum(m, lax.reduce_max(v, (0,)))

                    hit = lax.fori_loop(0, 8, _flt, jnp.int32(-1))

                    @pl.when(hit >= 0)
                    def _build_and_scatter():
                        @pl.loop(0, 8)
                        def _compute(i):
                            rows = i * 16 + lax.iota(jnp.int32, 16)
                            w16 = w_v[j, pl.ds(i * 16, 16)]
                            for jc in range(nch):
                                x = src_v[jc * 8 + j, pl.ds(i * 16, 16)] * w16
                                plsc.store_scatter(
                                    vals_v,
                                    [rows, jnp.full((16,), c0 + jc, jnp.int32)],
                                    x)
                            if with_metric:
                                plsc.store_scatter(
                                    vals_v,
                                    [rows, jnp.full((16,), C, jnp.int32)], w16)

                        pltpu.sync_copy(
                            vals_v,
                            table.at[plsc.Indices(idx128, ignored_value=-1)],
                            add=True)

        plsc.subcore_barrier()
        pltpu.sync_copy(
            table.at[pl.ds(pl.multiple_of(sid * TSUB, TSUB), TSUB), :],
            mid.at[n, pl.ds(pl.multiple_of(base + sid * TSUB, TSUB), TSUB), :])
        plsc.subcore_barrier()

    @pl.loop(cid, 2 * NPASS, step=NC)
    def _p(t):
        n = lax.div(t, jnp.int32(NPASS))
        base = (t - n * NPASS) * TROWS
        do_pass(n, base)


def _scatter_sc(inp, idx, w):
    mesh = plsc.VectorSubcoreMesh(
        core_axis_name="c", subcore_axis_name="s", num_cores=NC, num_subcores=NS)
    f = pl.kernel(
        _sc_body,
        out_type=jax.ShapeDtypeStruct((N, HW, GW), jnp.float32),
        mesh=mesh,
        compiler_params=pltpu.CompilerParams(needs_layout_passes=False),
        scratch_types=[
            pltpu.VMEM_SHARED((TROWS, GW), jnp.float32),
            pltpu.VMEM((8, 128), jnp.int32),
            pltpu.VMEM((8, 128), jnp.float32),
            pltpu.VMEM((17 * 8, 128), jnp.float32),
            pltpu.VMEM((128, GW), jnp.float32),
            pltpu.VMEM((128,), jnp.int32),
            pltpu.SemaphoreType.DMA,
        ],
    )
    zeros_hbm = jnp.zeros((TSUB, GW), jnp.float32)
    return f(inp.reshape(N, C, HW // 128, 128),
             idx.reshape(N * 4, HW // 128, 128),
             w.reshape(N * 4, HW // 128, 128),
             zeros_hbm)


def kernel(tenInput, tenFlow, tenMetric):
    idx, w = _prep(tenFlow, tenMetric)
    inp = tenInput.reshape(N, C, HW)
    mid = _scatter_sc(inp, idx, w)
    return _finalize(mid)
